# baseline (speedup 1.0000x reference)
import functools

import jax
import jax.numpy as jnp
from jax import lax
from jax.experimental import pallas as pl
from jax.experimental.pallas import tpu as pltpu
from jax.experimental.pallas import tpu_sc as plsc

N = 10000
E = 320000
D = 128
NC = 2
NS = 16
NW = NC * NS
EPW = E // NW
K = 112
NCHUNK = 90
NPAD = NCHUNK * K - EPW
NP = 10240
RPS = NP // NS

def _sc_agg_body(with_deg, h_hbm, eidx_hbm, *refs):
    if with_deg:
        out_hbm, deg_hbm = refs[0], refs[1]
        refs = refs[2:]
    else:
        out_hbm = refs[0]
        refs = refs[1:]
    (eb0, eb1, eb2, eb3,
     rows0, rows1, ones_v, zdeg, acc_sh, deg_sh,
     semi0, semi1, semi2, semi3, semg0, semg1, sems0, sems1) = refs
    ebuf = (eb0, eb1, eb2, eb3)
    rows = (rows0, rows1)
    semi = (semi0, semi1, semi2, semi3)
    semg = (semg0, semg1)
    sems = (sems0, sems1)

    c = lax.axis_index("c")
    s = lax.axis_index("s")
    wid = c * NS + s

    zero16 = jnp.zeros((16,), jnp.float32)

    def idx_issue(j, q):
        pltpu.async_copy(eidx_hbm.at[wid, j], ebuf[q], semi[q])

    def idx_wait(j, q):
        pltpu.make_async_copy(eidx_hbm.at[wid, j], ebuf[q], semi[q]).wait()

    def gather_issue(q, r):
        pltpu.async_copy(h_hbm.at[ebuf[q].at[0]], rows[r], semg[r])

    def gather_wait(q, r):
        pltpu.make_async_copy(h_hbm.at[ebuf[q].at[0]], rows[r], semg[r]).wait()

    def scatter_issue(q, r):
        pltpu.async_copy(rows[r], acc_sh.at[ebuf[q].at[1]], sems[r], add=True)
        if with_deg:
            pltpu.async_copy(ones_v, deg_sh.at[ebuf[q].at[1]], sems[r],
                             add=True)

    def scatter_wait(q, r):
        pltpu.make_async_copy(rows[r], acc_sh.at[ebuf[q].at[1]], sems[r]).wait()
        if with_deg:
            pltpu.make_async_copy(ones_v, deg_sh.at[ebuf[q].at[1]],
                                  sems[r]).wait()

    idx_issue(0, 0)
    idx_issue(1, 1)
    idx_issue(2, 2)

    def zrow(i, _):
        for jb in range(D // 16):
            rows1[i, pl.ds(jb * 16, 16)] = zero16
        return _

    lax.fori_loop(0, K, zrow, None)
    base = s * RPS
    off = 0
    while off < RPS:
        sz = min(K, RPS - off)
        pltpu.sync_copy(rows1.at[pl.ds(0, sz)], acc_sh.at[pl.ds(base + off, sz)])
        off += sz

    if with_deg:
        def zdrow(i, _):
            zdeg[pl.ds(i * 16, 16)] = zero16
            return _

        lax.fori_loop(0, 640 // 16, zdrow, None)
        pltpu.sync_copy(zdeg.at[pl.ds(0, RPS)], deg_sh.at[pl.ds(s * RPS, RPS)])
        one16 = jnp.full((16,), 1.0, jnp.float32)
        for jb in range(K // 16):
            ones_v[pl.ds(jb * 16, 16)] = one16

    idx_wait(0, 0)
    gather_issue(0, 0)
    plsc.subcore_barrier()

    def chunk_step(j, q, first):
        qn = (q + 1) % 4
        r = q % 2
        rn = qn % 2

        @pl.when(j + 1 < NCHUNK)
        def _():
            idx_wait(j + 1, qn)

        gather_wait(q, r)
        scatter_issue(q, r)
        if first:
            @pl.when(j > 0)
            def _():
                scatter_wait((q + 3) % 4, rn)
        else:
            scatter_wait((q + 3) % 4, rn)

        @pl.when(j + 1 < NCHUNK)
        def _():
            gather_issue(qn, rn)

        @pl.when(j + 3 < NCHUNK)
        def _():
            idx_issue(j + 3, (q + 3) % 4)

    def body(i, _):
        j = 4 * i
        chunk_step(j, 0, True)
        chunk_step(j + 1, 1, False)
        chunk_step(j + 2, 2, False)
        chunk_step(j + 3, 3, False)
        return _

    lax.fori_loop(0, NCHUNK // 4, body, None)
    for jj in range(4 * (NCHUNK // 4), NCHUNK):
        chunk_step(jj, jj % 4, False)
    scatter_wait((NCHUNK - 1) % 4, (NCHUNK - 1) % 2)

    plsc.subcore_barrier()

    pltpu.sync_copy(acc_sh.at[pl.ds(s * RPS, RPS)],
                    out_hbm.at[c, pl.ds(s * RPS, RPS)])
    if with_deg:
        pltpu.sync_copy(deg_sh.at[pl.ds(s * RPS, RPS)],
                        deg_hbm.at[c, pl.ds(s * RPS, RPS)])


def _sc_agg1d_body(s_hbm, eidx_hbm, out_hbm,
                   eidx_all, vals0, vals1, vals2, vals3, zdeg, acc_sh,
                   semg0, semg1, semg2, semg3, sems0, sems1, sems2, sems3):
    c = lax.axis_index("c")
    s = lax.axis_index("s")
    wid = c * NS + s
    vals = (vals0, vals1, vals2, vals3)
    semg = (semg0, semg1, semg2, semg3)
    sems = (sems0, sems1, sems2, sems3)

    zero16 = jnp.zeros((16,), jnp.float32)

    idx_cp = pltpu.async_copy(eidx_hbm.at[wid], eidx_all, semg0)

    def zdrow(i, _):
        zdeg[pl.ds(i * 16, 16)] = zero16
        return _

    lax.fori_loop(0, 640 // 16, zdrow, None)
    pltpu.sync_copy(zdeg.at[pl.ds(0, RPS)], acc_sh.at[pl.ds(s * RPS, RPS)])
    idx_cp.wait()
    plsc.subcore_barrier()

    def gather_issue(j, q):
        pltpu.async_copy(s_hbm.at[eidx_all.at[j, 0]], vals[q], semg[q])

    def gather_wait(j, q):
        pltpu.make_async_copy(s_hbm.at[eidx_all.at[j, 0]], vals[q],
                              semg[q]).wait()

    def scatter_issue(j, q):
        pltpu.async_copy(vals[q], acc_sh.at[eidx_all.at[j, 1]], sems[q],
                         add=True)

    def scatter_wait(j, q):
        pltpu.make_async_copy(vals[q], acc_sh.at[eidx_all.at[j, 1]],
                              sems[q]).wait()

    gather_issue(0, 0)

    def chunk_step(j, q):
        q1 = (q + 1) % 4
        gather_wait(j, q)
        scatter_issue(j, q)

        @pl.when(j >= 3)
        def _():
            scatter_wait(j - 3, q1)

        @pl.when(j + 1 < NCHUNK)
        def _():
            gather_issue(j + 1, q1)

    def body(i, _):
        j = 4 * i
        chunk_step(j, 0)
        chunk_step(j + 1, 1)
        chunk_step(j + 2, 2)
        chunk_step(j + 3, 3)
        return _

    lax.fori_loop(0, NCHUNK // 4, body, None)
    for jj in range(4 * (NCHUNK // 4), NCHUNK):
        chunk_step(jj, jj % 4)
    for jj in range(NCHUNK - 3, NCHUNK):
        scatter_wait(jj, jj % 4)

    plsc.subcore_barrier()
    pltpu.sync_copy(acc_sh.at[pl.ds(s * RPS, RPS)],
                    out_hbm.at[c, pl.ds(s * RPS, RPS)])


@functools.lru_cache(maxsize=None)
def _sc_kernels():
    mesh = plsc.VectorSubcoreMesh(
        core_axis_name="c", subcore_axis_name="s",
        num_cores=NC, num_subcores=NS,
    )
    agg_scratch = (
        [pltpu.VMEM((2, K), jnp.int32)] * 4
        + [pltpu.VMEM((K, D), jnp.float32)] * 2
        + [
            pltpu.VMEM((K,), jnp.float32),
            pltpu.VMEM((640,), jnp.float32),
            pltpu.VMEM_SHARED((NP, D), jnp.float32),
            pltpu.VMEM_SHARED((NP,), jnp.float32),
        ]
        + [pltpu.SemaphoreType.DMA] * 8
    )
    sc_agg_deg = pl.kernel(
        functools.partial(_sc_agg_body, True),
        out_type=[
            jax.ShapeDtypeStruct((NC, NP, D), jnp.float32),
            jax.ShapeDtypeStruct((NC, NP), jnp.float32),
        ],
        mesh=mesh,
        scratch_types=agg_scratch,
    )
    sc_agg = pl.kernel(
        functools.partial(_sc_agg_body, False),
        out_type=jax.ShapeDtypeStruct((NC, NP, D), jnp.float32),
        mesh=mesh,
        scratch_types=agg_scratch,
    )
    sc_agg1d = pl.kernel(
        _sc_agg1d_body,
        out_type=jax.ShapeDtypeStruct((NC, NP), jnp.float32),
        mesh=mesh,
        scratch_types=(
            [pltpu.VMEM((NCHUNK, 2, K), jnp.int32)]
            + [pltpu.VMEM((K,), jnp.float32)] * 4
            + [
                pltpu.VMEM((640,), jnp.float32),
                pltpu.VMEM_SHARED((NP,), jnp.float32),
            ]
            + [pltpu.SemaphoreType.DMA] * 8
        ),
    )
    return sc_agg_deg, sc_agg, sc_agg1d


BR = 2000
_GRID = N // BR


def _dot(a, w):
    return lax.dot_general(a, w, (((1,), (1,)), ((), ())),
                           preferred_element_type=jnp.float32)


def _tc_root_body(h_ref, wr_ref, bl_ref, root_ref):
    root_ref[...] = _dot(h_ref[...], wr_ref[...]) + bl_ref[...]


def _tc_comb1_body(p_ref, degp_ref, root_ref, wl_ref, h_ref, invdeg_ref):
    deg = degp_ref[0] + degp_ref[1]
    invdeg = 1.0 / jnp.maximum(deg, 1.0)
    agg = (p_ref[0] + p_ref[1]) * invdeg
    y = _dot(agg, wl_ref[...]) + root_ref[...]
    h_ref[...] = jnp.maximum(y, 0.0)
    invdeg_ref[...] = invdeg


def _tc_comb2_body(p_ref, invdeg_ref, root_ref, wl_ref, w3l_ref, w3r_ref,
                   s_ref, t_ref):
    agg = (p_ref[0] + p_ref[1]) * invdeg_ref[...]
    y = _dot(agg, wl_ref[...]) + root_ref[...]
    h2 = jnp.maximum(y, 0.0)
    s_ref[...] = lax.dot_general(h2, w3l_ref[...], (((1,), (1,)), ((), ())),
                                 preferred_element_type=jnp.float32)
    t_ref[...] = lax.dot_general(h2, w3r_ref[...], (((1,), (1,)), ((), ())),
                                 preferred_element_type=jnp.float32)


def _tc_layer3_body(ps_ref, invdeg_ref, t_ref, b_ref, out_ref):
    agg = (ps_ref[0] + ps_ref[1]) * invdeg_ref[...]
    out_ref[...] = agg + b_ref[...] + t_ref[...]


def _row_spec():
    return pl.BlockSpec((BR, D), lambda i: (i, 0))


def _col_spec():
    return pl.BlockSpec((BR, 1), lambda i: (i, 0))


def _full_spec():
    return pl.BlockSpec((D, D), lambda i: (0, 0))


def _bias_spec():
    return pl.BlockSpec((1, D), lambda i: (0, 0))


def _part_spec():
    return pl.BlockSpec((NC, BR, D), lambda i: (0, i, 0))


def _part1d_spec():
    return pl.BlockSpec((NC, BR, 1), lambda i: (0, i, 0))


def kernel(x, edge_index, W1l, b1l, W1r, W2l, b2l, W2r, W3l, b3l, W3r):
    pad_src = jnp.broadcast_to(
        ((jnp.arange(NPAD, dtype=jnp.int32) * 89) % N)[None], (NW, NPAD))
    pad_dst = jnp.broadcast_to(
        (N + jnp.arange(NPAD, dtype=jnp.int32))[None], (NW, NPAD))
    ew = edge_index.reshape(2, NW, EPW)
    eidx = jnp.concatenate(
        [ew, jnp.stack([pad_src, pad_dst])], axis=2).reshape(
            2, NW, NCHUNK, K).transpose(1, 2, 0, 3)
    _sc_agg_deg, _sc_agg, _sc_agg1d = _sc_kernels()

    def tc_root(h, Wr, bl):
        return pl.pallas_call(
            _tc_root_body,
            grid=(_GRID,),
            in_specs=[_row_spec(), _full_spec(), _bias_spec()],
            out_specs=_row_spec(),
            out_shape=jax.ShapeDtypeStruct((N, D), jnp.float32),
        )(h, Wr, bl.reshape(1, D))

    p1 = _sc_agg_deg(x, eidx)
    root1 = tc_root(x, W1r, b1l)
    p1, degp = p1

    h1, invdeg = pl.pallas_call(
        _tc_comb1_body,
        grid=(_GRID,),
        in_specs=[_part_spec(), _part1d_spec(), _row_spec(), _full_spec()],
        out_specs=[_row_spec(), _col_spec()],
        out_shape=[jax.ShapeDtypeStruct((N, D), jnp.float32),
                   jax.ShapeDtypeStruct((N, 1), jnp.float32)],
    )(p1, degp.reshape(NC, NP, 1), root1, W1l)

    p2 = _sc_agg(h1, eidx)
    root2 = tc_root(h1, W2r, b2l)

    s, t = pl.pallas_call(
        _tc_comb2_body,
        grid=(_GRID,),
        in_specs=[_part_spec(), _col_spec(), _row_spec(), _full_spec(),
                  pl.BlockSpec((1, D), lambda i: (0, 0)),
                  pl.BlockSpec((1, D), lambda i: (0, 0))],
        out_specs=[_col_spec(), _col_spec()],
        out_shape=[jax.ShapeDtypeStruct((N, 1), jnp.float32),
                   jax.ShapeDtypeStruct((N, 1), jnp.float32)],
    )(p2, invdeg, root2, W2l, W3l, W3r)

    ps = _sc_agg1d(s.reshape(N), eidx)

    out = pl.pallas_call(
        _tc_layer3_body,
        grid=(_GRID,),
        in_specs=[_part1d_spec(), _col_spec(), _col_spec(),
                  pl.BlockSpec((1, 1), lambda i: (0, 0))],
        out_specs=_col_spec(),
        out_shape=jax.ShapeDtypeStruct((N, 1), jnp.float32),
    )(ps.reshape(NC, NP, 1), invdeg, t, b3l.reshape(1, 1))

    return out


# speedup vs baseline: 1.0751x; 1.0751x over previous
import functools

import jax
import jax.numpy as jnp
from jax import lax
from jax.experimental import pallas as pl
from jax.experimental.pallas import tpu as pltpu
from jax.experimental.pallas import tpu_sc as plsc

N = 10000
E = 320000
D = 128
NC = 2
NS = 16
NW = NC * NS
EPW = E // NW
K = 112
NCHUNK = 90
NPAD = NCHUNK * K - EPW
NP = 10240
RPS = NP // NS

def _sc_agg_body(with_deg, h_hbm, eidx_hbm, *refs):
    if with_deg:
        out_hbm, deg_hbm = refs[0], refs[1]
        refs = refs[2:]
    else:
        out_hbm = refs[0]
        refs = refs[1:]
    (eb0, eb1, eb2, eb3,
     rows0, rows1, ones_v, zdeg, acc_sh, deg_sh,
     semi0, semi1, semi2, semi3, semg0, semg1, sems0, sems1) = refs
    ebuf = (eb0, eb1, eb2, eb3)
    rows = (rows0, rows1)
    semi = (semi0, semi1, semi2, semi3)
    semg = (semg0, semg1)
    sems = (sems0, sems1)

    c = lax.axis_index("c")
    s = lax.axis_index("s")
    wid = c * NS + s

    zero16 = jnp.zeros((16,), jnp.float32)

    def idx_issue(j, q):
        pltpu.async_copy(eidx_hbm.at[wid, j], ebuf[q], semi[q])

    def idx_wait(j, q):
        pltpu.make_async_copy(eidx_hbm.at[wid, j], ebuf[q], semi[q]).wait()

    def gather_issue(q, r):
        pltpu.async_copy(h_hbm.at[ebuf[q].at[0]], rows[r], semg[r])

    def gather_wait(q, r):
        pltpu.make_async_copy(h_hbm.at[ebuf[q].at[0]], rows[r], semg[r]).wait()

    def scatter_issue(q, r):
        pltpu.async_copy(rows[r], acc_sh.at[ebuf[q].at[1]], sems[r], add=True)
        if with_deg:
            pltpu.async_copy(ones_v, deg_sh.at[ebuf[q].at[1]], sems[r],
                             add=True)

    def scatter_wait(q, r):
        pltpu.make_async_copy(rows[r], acc_sh.at[ebuf[q].at[1]], sems[r]).wait()
        if with_deg:
            pltpu.make_async_copy(ones_v, deg_sh.at[ebuf[q].at[1]],
                                  sems[r]).wait()

    idx_issue(0, 0)
    idx_issue(1, 1)
    idx_issue(2, 2)

    def zrow(i, _):
        for jb in range(D // 16):
            rows1[i, pl.ds(jb * 16, 16)] = zero16
        return _

    lax.fori_loop(0, K, zrow, None)
    base = s * RPS
    off = 0
    while off < RPS:
        sz = min(K, RPS - off)
        pltpu.sync_copy(rows1.at[pl.ds(0, sz)], acc_sh.at[pl.ds(base + off, sz)])
        off += sz

    if with_deg:
        def zdrow(i, _):
            zdeg[pl.ds(i * 16, 16)] = zero16
            return _

        lax.fori_loop(0, 640 // 16, zdrow, None)
        pltpu.sync_copy(zdeg.at[pl.ds(0, RPS)], deg_sh.at[pl.ds(s * RPS, RPS)])
        one16 = jnp.full((16,), 1.0, jnp.float32)
        for jb in range(K // 16):
            ones_v[pl.ds(jb * 16, 16)] = one16

    idx_wait(0, 0)
    gather_issue(0, 0)
    plsc.subcore_barrier()

    def chunk_step(j, q, first):
        qn = (q + 1) % 4
        r = q % 2
        rn = qn % 2

        @pl.when(j + 1 < NCHUNK)
        def _():
            idx_wait(j + 1, qn)

        gather_wait(q, r)
        scatter_issue(q, r)
        if first:
            @pl.when(j > 0)
            def _():
                scatter_wait((q + 3) % 4, rn)
        else:
            scatter_wait((q + 3) % 4, rn)

        @pl.when(j + 1 < NCHUNK)
        def _():
            gather_issue(qn, rn)

        @pl.when(j + 3 < NCHUNK)
        def _():
            idx_issue(j + 3, (q + 3) % 4)

    def body(i, _):
        j = 4 * i
        chunk_step(j, 0, True)
        chunk_step(j + 1, 1, False)
        chunk_step(j + 2, 2, False)
        chunk_step(j + 3, 3, False)
        return _

    lax.fori_loop(0, NCHUNK // 4, body, None)
    for jj in range(4 * (NCHUNK // 4), NCHUNK):
        chunk_step(jj, jj % 4, False)
    scatter_wait((NCHUNK - 1) % 4, (NCHUNK - 1) % 2)

    plsc.subcore_barrier()

    pltpu.sync_copy(acc_sh.at[pl.ds(s * RPS, RPS)],
                    out_hbm.at[c, pl.ds(s * RPS, RPS)])
    if with_deg:
        pltpu.sync_copy(deg_sh.at[pl.ds(s * RPS, RPS)],
                        deg_hbm.at[c, pl.ds(s * RPS, RPS)])


def _sc_agg1d_body(s_hbm, eidx_hbm, out_hbm,
                   eidx_all, vals0, vals1, zdeg, acc_sh, sem0, sem1):
    c = lax.axis_index("c")
    s = lax.axis_index("s")
    wid = c * NS + s

    zero16 = jnp.zeros((16,), jnp.float32)

    idx_cp = pltpu.async_copy(eidx_hbm.at[wid], eidx_all, sem0)

    def zdrow(i, _):
        zdeg[pl.ds(i * 16, 16)] = zero16
        return _

    lax.fori_loop(0, 640 // 16, zdrow, None)
    pltpu.sync_copy(zdeg.at[pl.ds(0, RPS)], acc_sh.at[pl.ds(s * RPS, RPS)])
    idx_cp.wait()
    plsc.subcore_barrier()

    def gather(j, buf, sem):
        return pltpu.async_copy(s_hbm.at[eidx_all.at[j, 0]], buf, sem)

    def put(j, buf):
        pltpu.sync_copy(buf, acc_sh.at[eidx_all.at[j, 1]], add=True)

    def gwait(j, buf, sem):
        pltpu.make_async_copy(s_hbm.at[eidx_all.at[j, 0]], buf, sem).wait()

    gather(0, vals0, sem0)

    def body(i, _):
        j = 2 * i
        gather(j + 1, vals1, sem1)
        gwait(j, vals0, sem0)
        put(j, vals0)
        gather(j + 2, vals0, sem0)
        gwait(j + 1, vals1, sem1)
        put(j + 1, vals1)
        return _

    lax.fori_loop(0, NCHUNK // 2 - 1, body, None)
    gwait(NCHUNK - 2, vals0, sem0)
    put(NCHUNK - 2, vals0)
    gather(NCHUNK - 1, vals1, sem1)
    gwait(NCHUNK - 1, vals1, sem1)
    put(NCHUNK - 1, vals1)

    plsc.subcore_barrier()
    pltpu.sync_copy(acc_sh.at[pl.ds(s * RPS, RPS)],
                    out_hbm.at[c, pl.ds(s * RPS, RPS)])


@functools.lru_cache(maxsize=None)
def _sc_kernels():
    mesh = plsc.VectorSubcoreMesh(
        core_axis_name="c", subcore_axis_name="s",
        num_cores=NC, num_subcores=NS,
    )
    agg_scratch = (
        [pltpu.VMEM((2, K), jnp.int32)] * 4
        + [pltpu.VMEM((K, D), jnp.float32)] * 2
        + [
            pltpu.VMEM((K,), jnp.float32),
            pltpu.VMEM((640,), jnp.float32),
            pltpu.VMEM_SHARED((NP, D), jnp.float32),
            pltpu.VMEM_SHARED((NP,), jnp.float32),
        ]
        + [pltpu.SemaphoreType.DMA] * 8
    )
    sc_agg_deg = pl.kernel(
        functools.partial(_sc_agg_body, True),
        out_type=[
            jax.ShapeDtypeStruct((NC, NP, D), jnp.float32),
            jax.ShapeDtypeStruct((NC, NP), jnp.float32),
        ],
        mesh=mesh,
        scratch_types=agg_scratch,
    )
    sc_agg = pl.kernel(
        functools.partial(_sc_agg_body, False),
        out_type=jax.ShapeDtypeStruct((NC, NP, D), jnp.float32),
        mesh=mesh,
        scratch_types=agg_scratch,
    )
    sc_agg1d = pl.kernel(
        _sc_agg1d_body,
        out_type=jax.ShapeDtypeStruct((NC, NP), jnp.float32),
        mesh=mesh,
        scratch_types=[
            pltpu.VMEM((NCHUNK, 2, K), jnp.int32),
            pltpu.VMEM((K,), jnp.float32),
            pltpu.VMEM((K,), jnp.float32),
            pltpu.VMEM((640,), jnp.float32),
            pltpu.VMEM_SHARED((NP,), jnp.float32),
            pltpu.SemaphoreType.DMA,
            pltpu.SemaphoreType.DMA,
        ],
    )
    return sc_agg_deg, sc_agg, sc_agg1d


BR = 2000
_GRID = N // BR


def _dot(a, w):
    return lax.dot_general(a, w, (((1,), (1,)), ((), ())),
                           preferred_element_type=jnp.float32)


def _tc_root_body(h_ref, wr_ref, bl_ref, root_ref):
    root_ref[...] = _dot(h_ref[...], wr_ref[...]) + bl_ref[...]


def _tc_comb1_body(p_ref, degp_ref, root_ref, wl_ref, h_ref, invdeg_ref):
    deg = degp_ref[0] + degp_ref[1]
    invdeg = 1.0 / jnp.maximum(deg, 1.0)
    agg = (p_ref[0] + p_ref[1]) * invdeg
    y = _dot(agg, wl_ref[...]) + root_ref[...]
    h_ref[...] = jnp.maximum(y, 0.0)
    invdeg_ref[...] = invdeg


def _tc_comb2_body(p_ref, invdeg_ref, root_ref, wl_ref, w3l_ref, w3r_ref,
                   s_ref, t_ref):
    agg = (p_ref[0] + p_ref[1]) * invdeg_ref[...]
    y = _dot(agg, wl_ref[...]) + root_ref[...]
    h2 = jnp.maximum(y, 0.0)
    s_ref[...] = lax.dot_general(h2, w3l_ref[...], (((1,), (1,)), ((), ())),
                                 preferred_element_type=jnp.float32)
    t_ref[...] = lax.dot_general(h2, w3r_ref[...], (((1,), (1,)), ((), ())),
                                 preferred_element_type=jnp.float32)


def _tc_layer3_body(ps_ref, invdeg_ref, t_ref, b_ref, out_ref):
    agg = (ps_ref[0] + ps_ref[1]) * invdeg_ref[...]
    out_ref[...] = agg + b_ref[...] + t_ref[...]


def _row_spec():
    return pl.BlockSpec((BR, D), lambda i: (i, 0))


def _col_spec():
    return pl.BlockSpec((BR, 1), lambda i: (i, 0))


def _full_spec():
    return pl.BlockSpec((D, D), lambda i: (0, 0))


def _bias_spec():
    return pl.BlockSpec((1, D), lambda i: (0, 0))


def _part_spec():
    return pl.BlockSpec((NC, BR, D), lambda i: (0, i, 0))


def _part1d_spec():
    return pl.BlockSpec((NC, BR, 1), lambda i: (0, i, 0))


def kernel(x, edge_index, W1l, b1l, W1r, W2l, b2l, W2r, W3l, b3l, W3r):
    pad_src = jnp.broadcast_to(
        ((jnp.arange(NPAD, dtype=jnp.int32) * 89) % N)[None], (NW, NPAD))
    pad_dst = jnp.broadcast_to(
        (N + jnp.arange(NPAD, dtype=jnp.int32))[None], (NW, NPAD))
    ew = edge_index.reshape(2, NW, EPW)
    eidx = jnp.concatenate(
        [ew, jnp.stack([pad_src, pad_dst])], axis=2).reshape(
            2, NW, NCHUNK, K).transpose(1, 2, 0, 3)
    _sc_agg_deg, _sc_agg, _sc_agg1d = _sc_kernels()

    def tc_root(h, Wr, bl):
        return pl.pallas_call(
            _tc_root_body,
            grid=(_GRID,),
            in_specs=[_row_spec(), _full_spec(), _bias_spec()],
            out_specs=_row_spec(),
            out_shape=jax.ShapeDtypeStruct((N, D), jnp.float32),
        )(h, Wr, bl.reshape(1, D))

    p1 = _sc_agg_deg(x, eidx)
    root1 = tc_root(x, W1r, b1l)
    p1, degp = p1

    h1, invdeg = pl.pallas_call(
        _tc_comb1_body,
        grid=(_GRID,),
        in_specs=[_part_spec(), _part1d_spec(), _row_spec(), _full_spec()],
        out_specs=[_row_spec(), _col_spec()],
        out_shape=[jax.ShapeDtypeStruct((N, D), jnp.float32),
                   jax.ShapeDtypeStruct((N, 1), jnp.float32)],
    )(p1, degp.reshape(NC, NP, 1), root1, W1l)

    p2 = _sc_agg(h1, eidx)
    root2 = tc_root(h1, W2r, b2l)

    s, t = pl.pallas_call(
        _tc_comb2_body,
        grid=(_GRID,),
        in_specs=[_part_spec(), _col_spec(), _row_spec(), _full_spec(),
                  pl.BlockSpec((1, D), lambda i: (0, 0)),
                  pl.BlockSpec((1, D), lambda i: (0, 0))],
        out_specs=[_col_spec(), _col_spec()],
        out_shape=[jax.ShapeDtypeStruct((N, 1), jnp.float32),
                   jax.ShapeDtypeStruct((N, 1), jnp.float32)],
    )(p2, invdeg, root2, W2l, W3l, W3r)

    ps = _sc_agg1d(s.reshape(N), eidx)

    out = pl.pallas_call(
        _tc_layer3_body,
        grid=(_GRID,),
        in_specs=[_part1d_spec(), _col_spec(), _col_spec(),
                  pl.BlockSpec((1, 1), lambda i: (0, 0))],
        out_specs=_col_spec(),
        out_shape=jax.ShapeDtypeStruct((N, 1), jnp.float32),
    )(ps.reshape(NC, NP, 1), invdeg, t, b3l.reshape(1, 1))

    return out
